# baseline (speedup 1.0000x reference)
import functools

import jax
import jax.numpy as jnp
from jax import lax
from jax.experimental import pallas as pl
from jax.experimental.pallas import tpu as pltpu
from jax.experimental.pallas import tpu_sc as plsc

_B = 16384
_D = 64
_MARGIN = 1.0
_BC = 2048


def _rsqrt(x):
    bits = plsc.bitcast(x, jnp.int32)
    y = plsc.bitcast(jnp.int32(0x5F3759DF) - (bits >> 1), jnp.float32)
    for _ in range(3):
        y = y * (1.5 - 0.5 * x * y * y)
    return y


def _dist(a, b, u2, v2, uw, vw, ur, vr, uv, w2, r2, wr):
    ns2 = u2 + 2.0 * (a * uw) + (a * a) * w2
    nt2 = v2 + 2.0 * (b * vw) + (b * b) * w2
    ins = _rsqrt(jnp.maximum(ns2, 1e-24))
    int_ = _rsqrt(jnp.maximum(nt2, 1e-24))
    sr = ur + a * wr
    trr = vr + b * wr
    st = uv + b * uw + a * vw + (a * b) * w2
    d2 = 2.0 + r2 + 2.0 * (sr * ins - st * (ins * int_) - trr * int_)
    d2 = jnp.maximum(d2, 0.0)
    return d2 * _rsqrt(jnp.maximum(d2, 1e-30))


def _combine_body(x_ref, y_ref, o_ref):
    x = x_ref[...]
    y = y_ref[...]
    eye = jnp.eye(_D, dtype=jnp.float32)
    dn = (((0,), (0,)), ((), ()))
    xt = lax.dot_general(x, eye, dn, precision=lax.Precision.HIGHEST,
                         preferred_element_type=jnp.float32)
    yt = lax.dot_general(y, eye, dn, precision=lax.Precision.HIGHEST,
                         preferred_element_type=jnp.float32)
    g1 = jnp.sum(xt * yt, axis=1, keepdims=True)
    g2 = jnp.sum(xt * xt, axis=1, keepdims=True)
    right = jnp.concatenate(
        [g1, g2, jnp.zeros((xt.shape[0], 62), jnp.float32)], axis=1)
    o_ref[...] = jnp.concatenate([xt, right], axis=1)


def kernel(sp, tp, sn, tn, r, node_emb, link_emb, node_transfer, link_transfer):
    N = node_emb.shape[0]
    comb = pl.pallas_call(
        _combine_body,
        grid=(pl.cdiv(N, _BC),),
        in_specs=[pl.BlockSpec((_D, _BC), lambda i: (0, i)),
                  pl.BlockSpec((_D, _BC), lambda i: (0, i))],
        out_specs=pl.BlockSpec((_BC, 2 * _D), lambda i: (i, 0)),
        out_shape=jax.ShapeDtypeStruct((N, 2 * _D), jnp.float32),
    )(node_emb.T, node_transfer.T)

    info = plsc.get_sparse_core_info()
    NC, NS, L = info.num_cores, info.num_subcores, info.num_lanes
    NW = NC * NS
    PW = _B // NW
    C = 128
    NCH = PW // C
    G = C // L

    mesh = plsc.VectorSubcoreMesh(core_axis_name="c", subcore_axis_name="s")

    @functools.partial(
        pl.kernel,
        mesh=mesh,
        out_type=jax.ShapeDtypeStruct((NW, L), jnp.float32),
        compiler_params=pltpu.CompilerParams(
            needs_layout_passes=False, use_tc_tiling_on_sc=False),
        scratch_types=[
            pltpu.VMEM((PW,), jnp.int32),
            pltpu.VMEM((PW,), jnp.int32),
            pltpu.VMEM((PW,), jnp.int32),
            pltpu.VMEM((PW,), jnp.int32),
            pltpu.VMEM((PW,), jnp.int32),
            pltpu.VMEM((C, 2 * _D), jnp.float32),
            pltpu.VMEM((C, 2 * _D), jnp.float32),
            pltpu.VMEM((C, 2 * _D), jnp.float32),
            pltpu.VMEM((C, 2 * _D), jnp.float32),
            pltpu.VMEM((C, _D), jnp.float32),
            pltpu.VMEM((C, _D), jnp.float32),
            pltpu.VMEM((L,), jnp.float32),
            pltpu.SemaphoreType.DMA,
        ],
    )
    def tec(sp_h, tp_h, sn_h, tn_h, r_h, cb_h, le_h, lt_h, out_h,
            spv, tpv, snv, tnv, rv,
            bu, bv, bun, bvn, bw, bre,
            accv, sem):
        wid = lax.axis_index("s") * NC + lax.axis_index("c")
        base = pl.multiple_of(wid * PW, PW)
        pltpu.sync_copy(sp_h.at[pl.ds(base, PW)], spv)
        pltpu.sync_copy(tp_h.at[pl.ds(base, PW)], tpv)
        pltpu.sync_copy(sn_h.at[pl.ds(base, PW)], snv)
        pltpu.sync_copy(tn_h.at[pl.ds(base, PW)], tnv)
        pltpu.sync_copy(r_h.at[pl.ds(base, PW)], rv)

        lane = lax.iota(jnp.int32, L)
        zero = jnp.zeros((L,), jnp.float32)
        total = zero

        for c in range(NCH):
            off = c * C
            cps = [
                pltpu.async_copy(cb_h.at[spv.at[pl.ds(off, C)]], bu, sem),
                pltpu.async_copy(cb_h.at[tpv.at[pl.ds(off, C)]], bv, sem),
                pltpu.async_copy(cb_h.at[snv.at[pl.ds(off, C)]], bun, sem),
                pltpu.async_copy(cb_h.at[tnv.at[pl.ds(off, C)]], bvn, sem),
                pltpu.async_copy(lt_h.at[rv.at[pl.ds(off, C)]], bw, sem),
                pltpu.async_copy(le_h.at[rv.at[pl.ds(off, C)]], bre, sem),
            ]
            for cp in cps:
                cp.wait()

            def group_body(g, tot):
                rows = g * L + lane

                def dim_body(j, d):
                    col = (lane + j) & (_D - 1)
                    xu = plsc.load_gather(bu, [rows, col])
                    xv = plsc.load_gather(bv, [rows, col])
                    xun = plsc.load_gather(bun, [rows, col])
                    xvn = plsc.load_gather(bvn, [rows, col])
                    xw = plsc.load_gather(bw, [rows, col])
                    xr = plsc.load_gather(bre, [rows, col])
                    return (
                        d[0] + xu * xw, d[1] + xv * xw,
                        d[2] + xu * xr, d[3] + xv * xr,
                        d[4] + xu * xv,
                        d[5] + xun * xw, d[6] + xvn * xw,
                        d[7] + xun * xr, d[8] + xvn * xr,
                        d[9] + xun * xvn,
                        d[10] + xw * xw, d[11] + xr * xr,
                        d[12] + xw * xr,
                    )

                d = lax.fori_loop(0, _D, dim_body, (zero,) * 13)
                cg1 = jnp.zeros((L,), jnp.int32) + _D
                cg2 = cg1 + 1
                a = plsc.load_gather(bu, [rows, cg1])
                u2 = plsc.load_gather(bu, [rows, cg2])
                b = plsc.load_gather(bv, [rows, cg1])
                v2 = plsc.load_gather(bv, [rows, cg2])
                an = plsc.load_gather(bun, [rows, cg1])
                u2n = plsc.load_gather(bun, [rows, cg2])
                bn = plsc.load_gather(bvn, [rows, cg1])
                v2n = plsc.load_gather(bvn, [rows, cg2])
                pos = _dist(a, b, u2, v2, d[0], d[1], d[2], d[3], d[4],
                            d[10], d[11], d[12])
                neg = _dist(an, bn, u2n, v2n, d[5], d[6], d[7], d[8], d[9],
                            d[10], d[11], d[12])
                return tot + jnp.maximum(pos - neg + _MARGIN, 0.0)

            total = lax.fori_loop(0, G, group_body, total)

        accv[...] = total
        pltpu.sync_copy(accv, out_h.at[wid])

    parts = tec(sp.astype(jnp.int32), tp.astype(jnp.int32),
                sn.astype(jnp.int32), tn.astype(jnp.int32),
                r.astype(jnp.int32), comb, link_emb, link_transfer)
    return jnp.sum(parts) / _B


# speedup vs baseline: 3.7797x; 3.7797x over previous
import functools

import jax
import jax.numpy as jnp
from jax import lax
from jax.experimental import pallas as pl
from jax.experimental.pallas import tpu as pltpu
from jax.experimental.pallas import tpu_sc as plsc

_B = 16384
_D = 64
_MARGIN = 1.0
_BC = 2048


def _rsqrt(x):
    bits = plsc.bitcast(x, jnp.int32)
    y = plsc.bitcast(jnp.int32(0x5F3759DF) - (bits >> 1), jnp.float32)
    for _ in range(3):
        y = y * (1.5 - 0.5 * x * y * y)
    return y


def _dist(a, b, u2, v2, uw, vw, ur, vr, uv, w2, r2, wr):
    ns2 = u2 + 2.0 * (a * uw) + (a * a) * w2
    nt2 = v2 + 2.0 * (b * vw) + (b * b) * w2
    ins = _rsqrt(jnp.maximum(ns2, 1e-24))
    int_ = _rsqrt(jnp.maximum(nt2, 1e-24))
    sr = ur + a * wr
    trr = vr + b * wr
    st = uv + b * uw + a * vw + (a * b) * w2
    d2 = 2.0 + r2 + 2.0 * (sr * ins - st * (ins * int_) - trr * int_)
    d2 = jnp.maximum(d2, 0.0)
    return d2 * _rsqrt(jnp.maximum(d2, 1e-30))


def _combine_body(x_ref, y_ref, o_ref):
    x = x_ref[...]
    y = y_ref[...]
    x3 = jnp.concatenate([x, x * y, x * x], axis=0)
    h = x3.astype(jnp.bfloat16)
    m = (x3 - h.astype(jnp.float32)).astype(jnp.bfloat16)
    lhs = jnp.concatenate([h, m], axis=0)
    eye = jnp.eye(_D, dtype=jnp.float32)
    one = jnp.ones((_D, 1), jnp.float32)
    zc = jnp.zeros((_D, 1), jnp.float32)
    z64 = jnp.zeros((_D, _D), jnp.float32)
    r0 = jnp.concatenate([eye, zc, zc], axis=1)
    r1 = jnp.concatenate([z64, one, zc], axis=1)
    r2 = jnp.concatenate([z64, zc, one], axis=1)
    rhs = jnp.concatenate([r0, r1, r2, r0, r1, r2],
                          axis=0).astype(jnp.bfloat16)
    out = lax.dot_general(lhs, rhs, (((0,), (0,)), ((), ())),
                          preferred_element_type=jnp.float32)
    o_ref[:, : _D + 2] = out


def kernel(sp, tp, sn, tn, r, node_emb, link_emb, node_transfer, link_transfer):
    N = node_emb.shape[0]
    comb = pl.pallas_call(
        _combine_body,
        grid=(pl.cdiv(N, _BC),),
        in_specs=[pl.BlockSpec((_D, _BC), lambda i: (0, i)),
                  pl.BlockSpec((_D, _BC), lambda i: (0, i))],
        out_specs=pl.BlockSpec((_BC, 2 * _D), lambda i: (i, 0)),
        out_shape=jax.ShapeDtypeStruct((N, 2 * _D), jnp.float32),
    )(node_emb.T, node_transfer.T)

    info = plsc.get_sparse_core_info()
    NC, NS, L = info.num_cores, info.num_subcores, info.num_lanes
    NW = NC * NS
    PW = _B // NW
    C = 128
    NCH = PW // C
    G = C // L

    mesh = plsc.VectorSubcoreMesh(core_axis_name="c", subcore_axis_name="s")

    @functools.partial(
        pl.kernel,
        mesh=mesh,
        out_type=jax.ShapeDtypeStruct((NW, L), jnp.float32),
        compiler_params=pltpu.CompilerParams(
            needs_layout_passes=False, use_tc_tiling_on_sc=False),
        scratch_types=[
            pltpu.VMEM((PW,), jnp.int32),
            pltpu.VMEM((PW,), jnp.int32),
            pltpu.VMEM((PW,), jnp.int32),
            pltpu.VMEM((PW,), jnp.int32),
            pltpu.VMEM((PW,), jnp.int32),
            pltpu.VMEM((C, 2 * _D), jnp.float32),
            pltpu.VMEM((C, 2 * _D), jnp.float32),
            pltpu.VMEM((C, 2 * _D), jnp.float32),
            pltpu.VMEM((C, 2 * _D), jnp.float32),
            pltpu.VMEM((C, _D), jnp.float32),
            pltpu.VMEM((C, _D), jnp.float32),
            pltpu.VMEM((L,), jnp.float32),
            pltpu.SemaphoreType.DMA,
        ],
    )
    def tec(sp_h, tp_h, sn_h, tn_h, r_h, cb_h, le_h, lt_h, out_h,
            spv, tpv, snv, tnv, rv,
            bu, bv, bun, bvn, bw, bre,
            accv, sem):
        wid = lax.axis_index("s") * NC + lax.axis_index("c")
        base = pl.multiple_of(wid * PW, PW)
        pltpu.sync_copy(sp_h.at[pl.ds(base, PW)], spv)
        pltpu.sync_copy(tp_h.at[pl.ds(base, PW)], tpv)
        pltpu.sync_copy(sn_h.at[pl.ds(base, PW)], snv)
        pltpu.sync_copy(tn_h.at[pl.ds(base, PW)], tnv)
        pltpu.sync_copy(r_h.at[pl.ds(base, PW)], rv)

        lane = lax.iota(jnp.int32, L)
        zero = jnp.zeros((L,), jnp.float32)
        total = zero

        for c in range(NCH):
            off = c * C
            cps = [
                pltpu.async_copy(cb_h.at[spv.at[pl.ds(off, C)]], bu, sem),
                pltpu.async_copy(cb_h.at[tpv.at[pl.ds(off, C)]], bv, sem),
                pltpu.async_copy(cb_h.at[snv.at[pl.ds(off, C)]], bun, sem),
                pltpu.async_copy(cb_h.at[tnv.at[pl.ds(off, C)]], bvn, sem),
                pltpu.async_copy(lt_h.at[rv.at[pl.ds(off, C)]], bw, sem),
                pltpu.async_copy(le_h.at[rv.at[pl.ds(off, C)]], bre, sem),
            ]
            for cp in cps:
                cp.wait()

            def group_body(g, tot):
                rows = g * L + lane

                def dim_body(j, d):
                    col = (lane + j) & (_D - 1)
                    xu = plsc.load_gather(bu, [rows, col])
                    xv = plsc.load_gather(bv, [rows, col])
                    xun = plsc.load_gather(bun, [rows, col])
                    xvn = plsc.load_gather(bvn, [rows, col])
                    xw = plsc.load_gather(bw, [rows, col])
                    xr = plsc.load_gather(bre, [rows, col])
                    return (
                        d[0] + xu * xw, d[1] + xv * xw,
                        d[2] + xu * xr, d[3] + xv * xr,
                        d[4] + xu * xv,
                        d[5] + xun * xw, d[6] + xvn * xw,
                        d[7] + xun * xr, d[8] + xvn * xr,
                        d[9] + xun * xvn,
                        d[10] + xw * xw, d[11] + xr * xr,
                        d[12] + xw * xr,
                    )

                d = lax.fori_loop(0, _D, dim_body, (zero,) * 13)
                cg1 = jnp.zeros((L,), jnp.int32) + _D
                cg2 = cg1 + 1
                a = plsc.load_gather(bu, [rows, cg1])
                u2 = plsc.load_gather(bu, [rows, cg2])
                b = plsc.load_gather(bv, [rows, cg1])
                v2 = plsc.load_gather(bv, [rows, cg2])
                an = plsc.load_gather(bun, [rows, cg1])
                u2n = plsc.load_gather(bun, [rows, cg2])
                bn = plsc.load_gather(bvn, [rows, cg1])
                v2n = plsc.load_gather(bvn, [rows, cg2])
                pos = _dist(a, b, u2, v2, d[0], d[1], d[2], d[3], d[4],
                            d[10], d[11], d[12])
                neg = _dist(an, bn, u2n, v2n, d[5], d[6], d[7], d[8], d[9],
                            d[10], d[11], d[12])
                return tot + jnp.maximum(pos - neg + _MARGIN, 0.0)

            total = lax.fori_loop(0, G, group_body, total)

        accv[...] = total
        pltpu.sync_copy(accv, out_h.at[wid])

    parts = tec(sp.astype(jnp.int32), tp.astype(jnp.int32),
                sn.astype(jnp.int32), tn.astype(jnp.int32),
                r.astype(jnp.int32), comb, link_emb, link_transfer)
    return jnp.sum(parts) / _B
